# baseline (speedup 1.0000x reference)
import jax
import jax.numpy as jnp
from jax.experimental import pallas as pl
from jax.experimental.pallas import tpu as pltpu

_DIM = 4096
_EXPERTS = 64
_TOKENS = 8192
_TILE = 1024
_KS = 2
_KD = _DIM // _KS


def _gate_kernel(x_ref, wt_ref, b_ref, gs_ref, ts_ref, ti_ref, acc_ref):
    k = pl.program_id(1)
    part = jax.lax.dot_general(
        x_ref[...], wt_ref[...], (((1,), (0,)), ((), ())),
        preferred_element_type=jnp.float32,
    )

    @pl.when(k == 0)
    def _():
        acc_ref[...] = part

    @pl.when(k == _KS - 1)
    def _():
        logits = acc_ref[...] + part + b_ref[...]
        m = jnp.max(logits, axis=1, keepdims=True)
        e = jnp.exp(logits - m)
        s = jnp.sum(e, axis=1, keepdims=True)
        gate = e / s
        gs_ref[...] = gate
        idx = jax.lax.broadcasted_iota(jnp.int32, gate.shape, 1)
        m1 = jnp.max(gate, axis=1, keepdims=True)
        i1 = jnp.min(jnp.where(gate == m1, idx, _EXPERTS), axis=1, keepdims=True)
        masked = jnp.where(idx == i1, -jnp.inf, gate)
        m2 = jnp.max(masked, axis=1, keepdims=True)
        i2 = jnp.min(jnp.where(masked == m2, idx, _EXPERTS), axis=1, keepdims=True)
        ts_ref[...] = jnp.concatenate([m1, m2], axis=1)
        ti_ref[...] = jnp.concatenate([i1, i2], axis=1)


def kernel(x, W, b):
    wt = W.T
    b2 = b.reshape(1, _EXPERTS)
    grid = (_TOKENS // _TILE, _KS)
    out_shape = (
        jax.ShapeDtypeStruct((_TOKENS, _EXPERTS), jnp.float32),
        jax.ShapeDtypeStruct((_TOKENS, 2), jnp.float32),
        jax.ShapeDtypeStruct((_TOKENS, 2), jnp.int32),
    )
    gs, ts, ti = pl.pallas_call(
        _gate_kernel,
        grid=grid,
        in_specs=[
            pl.BlockSpec((_TILE, _KD), lambda i, k: (i, k)),
            pl.BlockSpec((_KD, _EXPERTS), lambda i, k: (k, 0)),
            pl.BlockSpec((1, _EXPERTS), lambda i, k: (0, 0)),
        ],
        out_specs=[
            pl.BlockSpec((_TILE, _EXPERTS), lambda i, k: (i, 0)),
            pl.BlockSpec((_TILE, 2), lambda i, k: (i, 0)),
            pl.BlockSpec((_TILE, 2), lambda i, k: (i, 0)),
        ],
        out_shape=out_shape,
        scratch_shapes=[pltpu.VMEM((_TILE, _EXPERTS), jnp.float32)],
    )(x, wt, b2)
    return (gs, ts, ti)


# speedup vs baseline: 1.1317x; 1.1317x over previous
import jax
import jax.numpy as jnp
from jax.experimental import pallas as pl

_DIM = 4096
_EXPERTS = 64
_TOKENS = 8192
_TILE = 1024
_NS = 4
_QD = _DIM // _NS


def _gate_kernel(*refs):
    x_refs = refs[:_NS]
    wt_refs = refs[_NS:2 * _NS]
    b_ref = refs[2 * _NS]
    gs_ref, ts_ref, ti_ref = refs[2 * _NS + 1:]
    dn = (((1,), (0,)), ((), ()))
    logits = b_ref[...]
    for xr, wr in zip(x_refs, wt_refs):
        logits = logits + jax.lax.dot_general(
            xr[...], wr[...], dn, preferred_element_type=jnp.float32
        )
    m = jnp.max(logits, axis=1, keepdims=True)
    e = jnp.exp(logits - m)
    s = jnp.sum(e, axis=1, keepdims=True)
    gate = e / s
    gs_ref[...] = gate

    idx = jax.lax.broadcasted_iota(jnp.int32, gate.shape, 1)
    m1 = jnp.max(gate, axis=1, keepdims=True)
    i1 = jnp.min(jnp.where(gate == m1, idx, _EXPERTS), axis=1, keepdims=True)
    masked = jnp.where(idx == i1, -jnp.inf, gate)
    m2 = jnp.max(masked, axis=1, keepdims=True)
    i2 = jnp.min(jnp.where(masked == m2, idx, _EXPERTS), axis=1, keepdims=True)
    ts_ref[...] = jnp.concatenate([m1, m2], axis=1)
    ti_ref[...] = jnp.concatenate([i1, i2], axis=1)


def kernel(x, W, b):
    wt = W.T
    b2 = b.reshape(1, _EXPERTS)
    grid = (_TOKENS // _TILE,)
    out_shape = (
        jax.ShapeDtypeStruct((_TOKENS, _EXPERTS), jnp.float32),
        jax.ShapeDtypeStruct((_TOKENS, 2), jnp.float32),
        jax.ShapeDtypeStruct((_TOKENS, 2), jnp.int32),
    )

    def mk_x(j):
        return pl.BlockSpec((_TILE, _QD), lambda i, j=j: (i, j))

    def mk_w(j):
        return pl.BlockSpec((_QD, _EXPERTS), lambda i, j=j: (j, 0))

    gs, ts, ti = pl.pallas_call(
        _gate_kernel,
        grid=grid,
        in_specs=[mk_x(j) for j in range(_NS)]
        + [mk_w(j) for j in range(_NS)]
        + [pl.BlockSpec((1, _EXPERTS), lambda i: (0, 0))],
        out_specs=[
            pl.BlockSpec((_TILE, _EXPERTS), lambda i: (i, 0)),
            pl.BlockSpec((_TILE, 2), lambda i: (i, 0)),
            pl.BlockSpec((_TILE, 2), lambda i: (i, 0)),
        ],
        out_shape=out_shape,
    )(*([x] * _NS), *([wt] * _NS), b2)
    return (gs, ts, ti)


# speedup vs baseline: 1.2085x; 1.0679x over previous
import jax
import jax.numpy as jnp
from jax.experimental import pallas as pl

_DIM = 4096
_EXPERTS = 64
_TOKENS = 8192
_TILE = 1024
_HD = _DIM // 2


def _gate_kernel(xa_ref, xb_ref, wa_ref, wb_ref, b_ref, gs_ref, ts_ref, ti_ref):
    dn = (((1,), (1,)), ((), ()))
    la = jax.lax.dot_general(
        xa_ref[...], wa_ref[...], dn, preferred_element_type=jnp.float32
    )
    lb = jax.lax.dot_general(
        xb_ref[...], wb_ref[...], dn, preferred_element_type=jnp.float32
    )
    logits = la + lb + b_ref[...]
    m = jnp.max(logits, axis=1, keepdims=True)
    e = jnp.exp(logits - m)
    s = jnp.sum(e, axis=1, keepdims=True)
    gate = e / s
    gs_ref[...] = gate

    idx = jax.lax.broadcasted_iota(jnp.int32, gate.shape, 1)
    m1 = jnp.max(gate, axis=1, keepdims=True)
    i1 = jnp.min(jnp.where(gate == m1, idx, _EXPERTS), axis=1, keepdims=True)
    masked = jnp.where(idx == i1, -jnp.inf, gate)
    m2 = jnp.max(masked, axis=1, keepdims=True)
    i2 = jnp.min(jnp.where(masked == m2, idx, _EXPERTS), axis=1, keepdims=True)
    ts_ref[...] = jnp.concatenate([m1, m2], axis=1)
    ti_ref[...] = jnp.concatenate([i1, i2], axis=1)


def kernel(x, W, b):
    b2 = b.reshape(1, _EXPERTS)
    grid = (_TOKENS // _TILE,)
    out_shape = (
        jax.ShapeDtypeStruct((_TOKENS, _EXPERTS), jnp.float32),
        jax.ShapeDtypeStruct((_TOKENS, 2), jnp.float32),
        jax.ShapeDtypeStruct((_TOKENS, 2), jnp.int32),
    )
    gs, ts, ti = pl.pallas_call(
        _gate_kernel,
        grid=grid,
        in_specs=[
            pl.BlockSpec((_TILE, _HD), lambda i: (i, 0)),
            pl.BlockSpec((_TILE, _HD), lambda i: (i, 1)),
            pl.BlockSpec((_EXPERTS, _HD), lambda i: (0, 0)),
            pl.BlockSpec((_EXPERTS, _HD), lambda i: (0, 1)),
            pl.BlockSpec((1, _EXPERTS), lambda i: (0, 0)),
        ],
        out_specs=[
            pl.BlockSpec((_TILE, _EXPERTS), lambda i: (i, 0)),
            pl.BlockSpec((_TILE, 2), lambda i: (i, 0)),
            pl.BlockSpec((_TILE, 2), lambda i: (i, 0)),
        ],
        out_shape=out_shape,
    )(x, x, W, W, b2)
    return (gs, ts, ti)


# speedup vs baseline: 1.2099x; 1.0011x over previous
import jax
import jax.numpy as jnp
from jax.experimental import pallas as pl

_DIM = 4096
_EXPERTS = 64
_TOKENS = 8192
_TILE = 1024
_HALF = _TILE // 2


def _top2(gate):
    idx = jax.lax.broadcasted_iota(jnp.int32, gate.shape, 1)
    m1 = jnp.max(gate, axis=1, keepdims=True)
    i1 = jnp.min(jnp.where(gate == m1, idx, _EXPERTS), axis=1, keepdims=True)
    masked = jnp.where(idx == i1, -jnp.inf, gate)
    m2 = jnp.max(masked, axis=1, keepdims=True)
    i2 = jnp.min(jnp.where(masked == m2, idx, _EXPERTS), axis=1, keepdims=True)
    return jnp.concatenate([m1, m2], axis=1), jnp.concatenate([i1, i2], axis=1)


def _gate_kernel(xa_ref, xb_ref, w_ref, b_ref, gs_ref, ts_ref, ti_ref):
    dn = (((1,), (1,)), ((), ()))
    w = w_ref[...]
    for half, xr in enumerate((xa_ref, xb_ref)):
        logits = jax.lax.dot_general(
            xr[...], w, dn, preferred_element_type=jnp.float32
        ) + b_ref[...]
        m = jnp.max(logits, axis=1, keepdims=True)
        e = jnp.exp(logits - m)
        s = jnp.sum(e, axis=1, keepdims=True)
        gate = e / s
        rows = pl.ds(half * _HALF, _HALF)
        gs_ref[rows, :] = gate
        ts, ti = _top2(gate)
        ts_ref[rows, :] = ts
        ti_ref[rows, :] = ti


def kernel(x, W, b):
    b2 = b.reshape(1, _EXPERTS)
    grid = (_TOKENS // _TILE,)
    out_shape = (
        jax.ShapeDtypeStruct((_TOKENS, _EXPERTS), jnp.float32),
        jax.ShapeDtypeStruct((_TOKENS, 2), jnp.float32),
        jax.ShapeDtypeStruct((_TOKENS, 2), jnp.int32),
    )
    gs, ts, ti = pl.pallas_call(
        _gate_kernel,
        grid=grid,
        in_specs=[
            pl.BlockSpec((_HALF, _DIM), lambda i: (2 * i, 0)),
            pl.BlockSpec((_HALF, _DIM), lambda i: (2 * i + 1, 0)),
            pl.BlockSpec((_EXPERTS, _DIM), lambda i: (0, 0)),
            pl.BlockSpec((1, _EXPERTS), lambda i: (0, 0)),
        ],
        out_specs=[
            pl.BlockSpec((_TILE, _EXPERTS), lambda i: (i, 0)),
            pl.BlockSpec((_TILE, 2), lambda i: (i, 0)),
            pl.BlockSpec((_TILE, 2), lambda i: (i, 0)),
        ],
        out_shape=out_shape,
    )(x, x, W, b2)
    return (gs, ts, ti)
